# baseline (speedup 1.0000x reference)
import functools

import jax
import jax.numpy as jnp
from jax import lax
from jax.experimental import pallas as pl
from jax.experimental.pallas import tpu as pltpu


def _topk2_weights(logits, n_exp):
    iota = lax.broadcasted_iota(jnp.int32, logits.shape, 1)
    v1 = jnp.max(logits, axis=1, keepdims=True)
    fi1 = jnp.min(jnp.where(logits == v1, iota, n_exp), axis=1, keepdims=True)
    m1 = iota == fi1
    neg = jnp.where(m1, -jnp.inf, logits)
    v2 = jnp.max(neg, axis=1, keepdims=True)
    fi2 = jnp.min(jnp.where(neg == v2, iota, n_exp), axis=1, keepdims=True)
    m2 = iota == fi2
    e2 = jnp.exp(v2 - v1)
    s1 = 1.0 / (1.0 + e2)
    s2 = e2 * s1
    zero = jnp.zeros_like(logits)
    return jnp.where(m1, s1, zero) + jnp.where(m2, s2, zero)


def _silu(x):
    return x * (1.0 / (1.0 + jnp.exp(-x)))


def _moe_body(x_ref, rw_ref, g_ref, u_ref, d_ref, out_ref, wts_ref, *, n_exp):
    e = pl.program_id(1)
    h = pl.program_id(2)
    first = jnp.logical_and(e == 0, h == 0)

    @pl.when(first)
    def _router():
        logits = jax.lax.dot_general(
            x_ref[...], rw_ref[...], (((1,), (1,)), ((), ())),
            preferred_element_type=jnp.float32)
        wts_ref[...] = _topk2_weights(logits, n_exp)

    x = x_ref[...].astype(jnp.bfloat16)
    g = jnp.dot(x, g_ref[0].astype(jnp.bfloat16),
                preferred_element_type=jnp.float32)
    u = jnp.dot(x, u_ref[0].astype(jnp.bfloat16),
                preferred_element_type=jnp.float32)
    o = jnp.dot((_silu(g) * u).astype(jnp.bfloat16),
                d_ref[0].astype(jnp.bfloat16),
                preferred_element_type=jnp.float32)
    wts = wts_ref[...]
    col = lax.broadcasted_iota(jnp.int32, wts.shape, 1) == e
    wcol = jnp.sum(jnp.where(col, wts, 0.0), axis=1, keepdims=True)
    contrib = wcol * o

    @pl.when(first)
    def _init():
        out_ref[...] = contrib

    @pl.when(jnp.logical_not(first))
    def _acc():
        out_ref[...] += contrib


def _ffn_body(x_ref, w1_ref, w3_ref, w2_ref, moe_ref, out_ref):
    s = pl.program_id(1)
    x = x_ref[...].astype(jnp.bfloat16)
    a = jax.lax.dot_general(x, w1_ref[...].astype(jnp.bfloat16),
                            (((1,), (1,)), ((), ())),
                            preferred_element_type=jnp.float32)
    b = jax.lax.dot_general(x, w3_ref[...].astype(jnp.bfloat16),
                            (((1,), (1,)), ((), ())),
                            preferred_element_type=jnp.float32)
    hblk = (_silu(a) * b).astype(jnp.bfloat16)
    o = jax.lax.dot_general(hblk, w2_ref[...].astype(jnp.bfloat16),
                            (((1,), (1,)), ((), ())),
                            preferred_element_type=jnp.float32)

    @pl.when(s == 0)
    def _init():
        out_ref[...] = moe_ref[...] + o

    @pl.when(s != 0)
    def _acc():
        out_ref[...] += o


def kernel(x, router_w, gate_proj, up_proj, down_proj, w1, w2, w3):
    T, D = x.shape
    E, _, H = gate_proj.shape
    SH = w1.shape[0]

    TM = min(T, 1024)
    HC = min(H, 256)
    TMF = min(T, 512)
    SC = min(SH, 512)
    t2, h2 = T // TM, H // HC
    tf2, s2 = T // TMF, SH // SC

    moe_out = pl.pallas_call(
        functools.partial(_moe_body, n_exp=E),
        grid=(t2, E, h2),
        in_specs=[
            pl.BlockSpec((TM, D), lambda t, e, h: (t, 0)),
            pl.BlockSpec((E, D), lambda t, e, h: (0, 0)),
            pl.BlockSpec((1, D, HC), lambda t, e, h: (e, 0, h)),
            pl.BlockSpec((1, D, HC), lambda t, e, h: (e, 0, h)),
            pl.BlockSpec((1, HC, D), lambda t, e, h: (e, h, 0)),
        ],
        out_specs=pl.BlockSpec((TM, D), lambda t, e, h: (t, 0)),
        out_shape=jax.ShapeDtypeStruct((T, D), jnp.float32),
        scratch_shapes=[pltpu.VMEM((TM, E), jnp.float32)],
    )(x, router_w, gate_proj, up_proj, down_proj)

    out = pl.pallas_call(
        _ffn_body,
        grid=(tf2, s2),
        in_specs=[
            pl.BlockSpec((TMF, D), lambda t, s: (t, 0)),
            pl.BlockSpec((SC, D), lambda t, s: (s, 0)),
            pl.BlockSpec((SC, D), lambda t, s: (s, 0)),
            pl.BlockSpec((D, SC), lambda t, s: (0, s)),
            pl.BlockSpec((TMF, D), lambda t, s: (t, 0)),
        ],
        out_specs=pl.BlockSpec((TMF, D), lambda t, s: (t, 0)),
        out_shape=jax.ShapeDtypeStruct((T, D), jnp.float32),
    )(x, w1, w3, w2, moe_out)

    return out


# speedup vs baseline: 1.1971x; 1.1971x over previous
import functools

import jax
import jax.numpy as jnp
from jax import lax
from jax.experimental import pallas as pl
from jax.experimental.pallas import tpu as pltpu
from jax.experimental.pallas import tpu_sc as plsc

TMG = 128


def _silu(x):
    return x * (1.0 / (1.0 + jnp.exp(-x)))


def _tri(n, dtype=jnp.float32):
    r = lax.broadcasted_iota(jnp.int32, (n, n), 0)
    c = lax.broadcasted_iota(jnp.int32, (n, n), 1)
    return (r > c).astype(dtype)


def _plan_body(x_ref, rw_ref, ipos_ref, scores_ref, te_ref, *, n_exp, n_tiles):
    T = x_ref.shape[0]
    logits = lax.dot_general(x_ref[...], rw_ref[...], (((1,), (1,)), ((), ())),
                             preferred_element_type=jnp.float32)
    iota = lax.broadcasted_iota(jnp.int32, logits.shape, 1)
    v1 = jnp.max(logits, axis=1, keepdims=True)
    fi1 = jnp.min(jnp.where(logits == v1, iota, n_exp), axis=1, keepdims=True)
    m1 = iota == fi1
    neg = jnp.where(m1, -jnp.inf, logits)
    v2 = jnp.max(neg, axis=1, keepdims=True)
    fi2 = jnp.min(jnp.where(neg == v2, iota, n_exp), axis=1, keepdims=True)
    m2 = iota == fi2
    e2 = jnp.exp(v2 - v1)
    s1 = 1.0 / (1.0 + e2)
    s2 = e2 * s1

    M0 = m1.astype(jnp.float32)
    M1 = m2.astype(jnp.float32)
    S = M0 + M1

    nb, bs = T // 128, 128
    S3 = S.reshape(nb, bs, n_exp)
    L = jnp.broadcast_to(_tri(bs), (nb, bs, bs))
    P3 = lax.dot_general(L, S3, (((2,), (1,)), ((0,), (0,))),
                         preferred_element_type=jnp.float32)
    BS = jnp.sum(S3, axis=1)
    BP = lax.dot_general(_tri(nb), BS, (((1,), (0,)), ((), ())),
                         preferred_element_type=jnp.float32)
    C = (P3 + BP[:, None, :]).reshape(T, n_exp)

    hist = jnp.sum(S, axis=0, keepdims=True)
    histc = jnp.floor((hist + (TMG - 1)) * (1.0 / TMG)) * TMG
    U = (lax.broadcasted_iota(jnp.int32, (n_exp, n_exp), 0) <
         lax.broadcasted_iota(jnp.int32, (n_exp, n_exp), 1)).astype(jnp.float32)
    start = lax.dot_general(histc, U, (((1,), (0,)), ((), ())),
                            preferred_element_type=jnp.float32)

    pos0 = jnp.sum((start + C) * M0, axis=1, keepdims=True)
    pos1 = jnp.sum((start + C) * M1, axis=1, keepdims=True)
    ipos_ref[...] = jnp.concatenate([pos0, pos1], axis=1).astype(jnp.int32)
    scores_ref[...] = jnp.concatenate([s1, s2], axis=1)

    ti = lax.broadcasted_iota(
        jnp.int32, (n_tiles, n_exp), 0).astype(jnp.float32) * TMG
    startb = jnp.broadcast_to(start, (n_tiles, n_exp))
    te_ref[...] = (jnp.sum((startb <= ti).astype(jnp.int32), axis=1,
                           keepdims=True) - 1)


def _gmm_body(te_ref, xs_ref, g_ref, u_ref, d_ref, os_ref):
    xt = xs_ref[...]
    g = jnp.dot(xt, g_ref[0], preferred_element_type=jnp.float32)
    u = jnp.dot(xt, u_ref[0], preferred_element_type=jnp.float32)
    os_ref[...] = jnp.dot(_silu(g) * u, d_ref[0],
                          preferred_element_type=jnp.float32)


def _ffn_body(x_ref, w1_ref, w3_ref, w2_ref, out_ref):
    s = pl.program_id(1)
    x = x_ref[...]
    a = lax.dot_general(x, w1_ref[...], (((1,), (1,)), ((), ())),
                        preferred_element_type=jnp.float32)
    b = lax.dot_general(x, w3_ref[...], (((1,), (1,)), ((), ())),
                        preferred_element_type=jnp.float32)
    hblk = _silu(a) * b
    o = lax.dot_general(hblk, w2_ref[...], (((1,), (1,)), ((), ())),
                        preferred_element_type=jnp.float32)

    @pl.when(s == 0)
    def _init():
        out_ref[...] = o

    @pl.when(s != 0)
    def _acc():
        out_ref[...] += o


def _make_dispatch(T, D, PAD, nw, ch):
    tpw = T // nw
    nch = tpw // ch
    mesh = plsc.VectorSubcoreMesh(core_axis_name="c", subcore_axis_name="s", num_cores=2, num_subcores=16)

    @functools.partial(
        pl.kernel,
        out_type=jax.ShapeDtypeStruct((PAD, D), jnp.float32),
        mesh=mesh,
        scratch_types=[
            pltpu.VMEM((ch,), jnp.int32),
            pltpu.VMEM((ch,), jnp.int32),
            pltpu.VMEM((ch, D), jnp.float32),
            pltpu.SemaphoreType.DMA,
        ],
    )
    def dispatch(x_hbm, i0_hbm, i1_hbm, xs_hbm, idx0_v, idx1_v, rows_v, sem):
        nc = jax.lax.axis_size("c")
        wid = lax.axis_index("s") * nc + lax.axis_index("c")
        base = wid * tpw

        def chunk(j, carry):
            tb = pl.multiple_of(base + j * ch, 8)
            pltpu.sync_copy(x_hbm.at[pl.ds(tb, ch)], rows_v)
            pltpu.sync_copy(i0_hbm.at[pl.ds(tb, ch)], idx0_v)
            pltpu.sync_copy(i1_hbm.at[pl.ds(tb, ch)], idx1_v)
            pltpu.async_copy(rows_v, xs_hbm.at[idx0_v], sem).wait()
            pltpu.async_copy(rows_v, xs_hbm.at[idx1_v], sem).wait()
            return carry

        lax.fori_loop(0, nch, chunk, 0)

    return dispatch


def _make_combine(T, D, PAD, nw, ch):
    tpw = T // nw
    nch = tpw // ch
    mesh = plsc.VectorSubcoreMesh(core_axis_name="c", subcore_axis_name="s", num_cores=2, num_subcores=16)

    @functools.partial(
        pl.kernel,
        out_type=(jax.ShapeDtypeStruct((T, D), jnp.float32),
                  jax.ShapeDtypeStruct((T, D), jnp.float32)),
        mesh=mesh,
        scratch_types=[
            pltpu.VMEM((ch,), jnp.int32),
            pltpu.VMEM((ch,), jnp.int32),
            pltpu.VMEM((ch, D), jnp.float32),
            pltpu.VMEM((ch, D), jnp.float32),
            pltpu.SemaphoreType.DMA,
        ],
    )
    def combine(os_hbm, i0_hbm, i1_hbm, oc0_hbm, oc1_hbm,
                idx0_v, idx1_v, a_v, b_v, sem):
        nc = jax.lax.axis_size("c")
        wid = lax.axis_index("s") * nc + lax.axis_index("c")
        base = wid * tpw

        def chunk(j, carry):
            tb = pl.multiple_of(base + j * ch, 8)
            pltpu.sync_copy(i0_hbm.at[pl.ds(tb, ch)], idx0_v)
            pltpu.sync_copy(i1_hbm.at[pl.ds(tb, ch)], idx1_v)
            pltpu.async_copy(os_hbm.at[idx0_v], a_v, sem).wait()
            pltpu.async_copy(os_hbm.at[idx1_v], b_v, sem).wait()
            pltpu.sync_copy(a_v, oc0_hbm.at[pl.ds(tb, ch)])
            pltpu.sync_copy(b_v, oc1_hbm.at[pl.ds(tb, ch)])
            return carry

        lax.fori_loop(0, nch, chunk, 0)

    return combine


def _scale_body(oc0_ref, oc1_ref, s0_ref, s1_ref, ffn_ref, out_ref):
    out_ref[...] = (s0_ref[...] * oc0_ref[...] + s1_ref[...] * oc1_ref[...]
                    + ffn_ref[...])


def kernel(x, router_w, gate_proj, up_proj, down_proj, w1, w2, w3):
    T, D = x.shape
    E, _, H = gate_proj.shape
    SH = w1.shape[0]
    K = 2
    PAD = T * K + E * TMG
    NT = PAD // TMG

    ipos, scores, te = pl.pallas_call(
        functools.partial(_plan_body, n_exp=E, n_tiles=NT),
        grid=(1,),
        in_specs=[
            pl.BlockSpec((T, D), lambda i: (0, 0)),
            pl.BlockSpec((E, D), lambda i: (0, 0)),
        ],
        out_specs=[
            pl.BlockSpec((T, K), lambda i: (0, 0)),
            pl.BlockSpec((T, K), lambda i: (0, 0)),
            pl.BlockSpec((NT, 1), lambda i: (0, 0)),
        ],
        out_shape=[
            jax.ShapeDtypeStruct((T, K), jnp.int32),
            jax.ShapeDtypeStruct((T, K), jnp.float32),
            jax.ShapeDtypeStruct((NT, 1), jnp.int32),
        ],
    )(x, router_w)

    ipos0 = ipos[:, 0]
    ipos1 = ipos[:, 1]
    te_flat = te.reshape(NT)

    nw, ch = 32, 16
    xs = _make_dispatch(T, D, PAD, nw, ch)(x, ipos0, ipos1)

    grid_spec = pltpu.PrefetchScalarGridSpec(
        num_scalar_prefetch=1,
        grid=(NT,),
        in_specs=[
            pl.BlockSpec((TMG, D), lambda i, te_r: (i, 0)),
            pl.BlockSpec((1, D, H), lambda i, te_r: (te_r[i], 0, 0)),
            pl.BlockSpec((1, D, H), lambda i, te_r: (te_r[i], 0, 0)),
            pl.BlockSpec((1, H, D), lambda i, te_r: (te_r[i], 0, 0)),
        ],
        out_specs=pl.BlockSpec((TMG, D), lambda i, te_r: (i, 0)),
    )
    os_rows = pl.pallas_call(
        _gmm_body,
        grid_spec=grid_spec,
        out_shape=jax.ShapeDtypeStruct((PAD, D), jnp.float32),
    )(te_flat, xs, gate_proj, up_proj, down_proj)

    oc0, oc1 = _make_combine(T, D, PAD, nw, ch)(os_rows, ipos0, ipos1)

    TMF = min(T, 1024)
    SC = min(SH, 128)
    tf2, sh2 = T // TMF, SH // SC
    ffn = pl.pallas_call(
        _ffn_body,
        grid=(tf2, sh2),
        in_specs=[
            pl.BlockSpec((TMF, D), lambda t, s: (t, 0)),
            pl.BlockSpec((SC, D), lambda t, s: (s, 0)),
            pl.BlockSpec((SC, D), lambda t, s: (s, 0)),
            pl.BlockSpec((D, SC), lambda t, s: (0, s)),
        ],
        out_specs=pl.BlockSpec((TMF, D), lambda t, s: (t, 0)),
        out_shape=jax.ShapeDtypeStruct((T, D), jnp.float32),
    )(x, w1, w3, w2)

    s0c = scores[:, 0:1]
    s1c = scores[:, 1:2]
    TMS = min(T, 512)
    out = pl.pallas_call(
        _scale_body,
        grid=(T // TMS,),
        in_specs=[
            pl.BlockSpec((TMS, D), lambda t: (t, 0)),
            pl.BlockSpec((TMS, D), lambda t: (t, 0)),
            pl.BlockSpec((TMS, 1), lambda t: (t, 0)),
            pl.BlockSpec((TMS, 1), lambda t: (t, 0)),
            pl.BlockSpec((TMS, D), lambda t: (t, 0)),
        ],
        out_specs=pl.BlockSpec((TMS, D), lambda t: (t, 0)),
        out_shape=jax.ShapeDtypeStruct((T, D), jnp.float32),
    )(oc0, oc1, s0c, s1c, ffn)

    return out


# speedup vs baseline: 1.3812x; 1.1538x over previous
import functools

import jax
import jax.numpy as jnp
from jax import lax
from jax.experimental import pallas as pl
from jax.experimental.pallas import tpu as pltpu
from jax.experimental.pallas import tpu_sc as plsc

TMG = 128


def _silu(x):
    return x * (1.0 / (1.0 + jnp.exp(-x)))


def _tri(n, dtype=jnp.float32):
    r = lax.broadcasted_iota(jnp.int32, (n, n), 0)
    c = lax.broadcasted_iota(jnp.int32, (n, n), 1)
    return (r > c).astype(dtype)


def _plan_body(x_ref, rw_ref, ipos_ref, scores_ref, te_ref, *, n_exp, n_tiles):
    T = x_ref.shape[0]
    logits = lax.dot_general(x_ref[...], rw_ref[...], (((1,), (1,)), ((), ())),
                             preferred_element_type=jnp.float32)
    iota = lax.broadcasted_iota(jnp.int32, logits.shape, 1)
    v1 = jnp.max(logits, axis=1, keepdims=True)
    fi1 = jnp.min(jnp.where(logits == v1, iota, n_exp), axis=1, keepdims=True)
    m1 = iota == fi1
    neg = jnp.where(m1, -jnp.inf, logits)
    v2 = jnp.max(neg, axis=1, keepdims=True)
    fi2 = jnp.min(jnp.where(neg == v2, iota, n_exp), axis=1, keepdims=True)
    m2 = iota == fi2
    e2 = jnp.exp(v2 - v1)
    s1 = 1.0 / (1.0 + e2)
    s2 = e2 * s1

    M0 = m1.astype(jnp.float32)
    M1 = m2.astype(jnp.float32)
    S = M0 + M1

    nb, bs = T // 128, 128
    S3 = S.reshape(nb, bs, n_exp)
    L = jnp.broadcast_to(_tri(bs), (nb, bs, bs))
    P3 = lax.dot_general(L, S3, (((2,), (1,)), ((0,), (0,))),
                         preferred_element_type=jnp.float32)
    BS = jnp.sum(S3, axis=1)
    BP = lax.dot_general(_tri(nb), BS, (((1,), (0,)), ((), ())),
                         preferred_element_type=jnp.float32)
    C = (P3 + BP[:, None, :]).reshape(T, n_exp)

    hist = jnp.sum(S, axis=0, keepdims=True)
    histc = jnp.floor((hist + (TMG - 1)) * (1.0 / TMG)) * TMG
    U = (lax.broadcasted_iota(jnp.int32, (n_exp, n_exp), 0) <
         lax.broadcasted_iota(jnp.int32, (n_exp, n_exp), 1)).astype(jnp.float32)
    start = lax.dot_general(histc, U, (((1,), (0,)), ((), ())),
                            preferred_element_type=jnp.float32)

    pos0 = jnp.sum((start + C) * M0, axis=1, keepdims=True)
    pos1 = jnp.sum((start + C) * M1, axis=1, keepdims=True)
    ipos_ref[...] = jnp.concatenate([pos0, pos1], axis=1).astype(jnp.int32)
    scores_ref[...] = jnp.concatenate([s1, s2], axis=1)

    ti = lax.broadcasted_iota(
        jnp.int32, (n_tiles, n_exp), 0).astype(jnp.float32) * TMG
    startb = jnp.broadcast_to(start, (n_tiles, n_exp))
    te_ref[...] = (jnp.sum((startb <= ti).astype(jnp.int32), axis=1,
                           keepdims=True) - 1)


def _gmm_body(te_ref, xs_ref, g_ref, u_ref, d_ref, os_ref):
    xt = xs_ref[...]
    g = jnp.dot(xt, g_ref[0], preferred_element_type=jnp.float32)
    u = jnp.dot(xt, u_ref[0], preferred_element_type=jnp.float32)
    os_ref[...] = jnp.dot(_silu(g) * u, d_ref[0],
                          preferred_element_type=jnp.float32)


def _ffn_body(x_ref, w1_ref, w3_ref, w2_ref, out_ref):
    s = pl.program_id(1)
    x = x_ref[...]
    a = lax.dot_general(x, w1_ref[...], (((1,), (1,)), ((), ())),
                        preferred_element_type=jnp.float32)
    b = lax.dot_general(x, w3_ref[...], (((1,), (1,)), ((), ())),
                        preferred_element_type=jnp.float32)
    hblk = _silu(a) * b
    o = lax.dot_general(hblk, w2_ref[...], (((1,), (1,)), ((), ())),
                        preferred_element_type=jnp.float32)

    @pl.when(s == 0)
    def _init():
        out_ref[...] = o

    @pl.when(s != 0)
    def _acc():
        out_ref[...] += o


def _make_dispatch(T, D, PAD, nw, ch):
    tpw = T // nw
    nch = tpw // ch
    mesh = plsc.VectorSubcoreMesh(core_axis_name="c", subcore_axis_name="s", num_cores=2, num_subcores=16)

    @functools.partial(
        pl.kernel,
        out_type=jax.ShapeDtypeStruct((PAD, D), jnp.float32),
        mesh=mesh,
        scratch_types=[
            pltpu.VMEM((ch,), jnp.int32),
            pltpu.VMEM((ch,), jnp.int32),
            pltpu.VMEM((ch, D), jnp.float32),
            pltpu.SemaphoreType.DMA,
        ],
    )
    def dispatch(x_hbm, i0_hbm, i1_hbm, xs_hbm, idx0_v, idx1_v, rows_v, sem):
        nc = jax.lax.axis_size("c")
        wid = lax.axis_index("s") * nc + lax.axis_index("c")
        base = wid * tpw

        def chunk(j, carry):
            tb = pl.multiple_of(base + j * ch, 8)
            pltpu.sync_copy(x_hbm.at[pl.ds(tb, ch)], rows_v)
            pltpu.sync_copy(i0_hbm.at[pl.ds(tb, ch)], idx0_v)
            pltpu.sync_copy(i1_hbm.at[pl.ds(tb, ch)], idx1_v)
            pltpu.async_copy(rows_v, xs_hbm.at[idx0_v], sem).wait()
            pltpu.async_copy(rows_v, xs_hbm.at[idx1_v], sem).wait()
            return carry

        lax.fori_loop(0, nch, chunk, 0)

    return dispatch


def _make_combine(T, D, PAD, nw, ch):
    tpw = T // nw
    nch = tpw // ch
    mesh = plsc.VectorSubcoreMesh(core_axis_name="c", subcore_axis_name="s", num_cores=2, num_subcores=16)

    @functools.partial(
        pl.kernel,
        out_type=(jax.ShapeDtypeStruct((T, D), jnp.float32),
                  jax.ShapeDtypeStruct((T, D), jnp.float32)),
        mesh=mesh,
        scratch_types=[
            pltpu.VMEM((ch,), jnp.int32),
            pltpu.VMEM((ch,), jnp.int32),
            pltpu.VMEM((ch, D), jnp.float32),
            pltpu.VMEM((ch, D), jnp.float32),
            pltpu.SemaphoreType.DMA,
        ],
    )
    def combine(os_hbm, i0_hbm, i1_hbm, oc0_hbm, oc1_hbm,
                idx0_v, idx1_v, a_v, b_v, sem):
        nc = jax.lax.axis_size("c")
        wid = lax.axis_index("s") * nc + lax.axis_index("c")
        base = wid * tpw

        def chunk(j, carry):
            tb = pl.multiple_of(base + j * ch, 8)
            pltpu.sync_copy(i0_hbm.at[pl.ds(tb, ch)], idx0_v)
            pltpu.sync_copy(i1_hbm.at[pl.ds(tb, ch)], idx1_v)
            pltpu.async_copy(os_hbm.at[idx0_v], a_v, sem).wait()
            pltpu.async_copy(os_hbm.at[idx1_v], b_v, sem).wait()
            pltpu.sync_copy(a_v, oc0_hbm.at[pl.ds(tb, ch)])
            pltpu.sync_copy(b_v, oc1_hbm.at[pl.ds(tb, ch)])
            return carry

        lax.fori_loop(0, nch, chunk, 0)

    return combine


def _scale_body(oc0_ref, oc1_ref, s0_ref, s1_ref, ffn_ref, out_ref):
    out_ref[...] = (s0_ref[...] * oc0_ref[...] + s1_ref[...] * oc1_ref[...]
                    + ffn_ref[...])


def kernel(x, router_w, gate_proj, up_proj, down_proj, w1, w2, w3):
    T, D = x.shape
    E, _, H = gate_proj.shape
    SH = w1.shape[0]
    K = 2
    PAD = T * K + E * TMG
    NT = PAD // TMG

    ipos, scores, te = pl.pallas_call(
        functools.partial(_plan_body, n_exp=E, n_tiles=NT),
        grid=(1,),
        in_specs=[
            pl.BlockSpec((T, D), lambda i: (0, 0)),
            pl.BlockSpec((E, D), lambda i: (0, 0)),
        ],
        out_specs=[
            pl.BlockSpec((T, K), lambda i: (0, 0)),
            pl.BlockSpec((T, K), lambda i: (0, 0)),
            pl.BlockSpec((NT, 1), lambda i: (0, 0)),
        ],
        out_shape=[
            jax.ShapeDtypeStruct((T, K), jnp.int32),
            jax.ShapeDtypeStruct((T, K), jnp.float32),
            jax.ShapeDtypeStruct((NT, 1), jnp.int32),
        ],
    )(x, router_w)

    ipos0 = ipos[:, 0]
    ipos1 = ipos[:, 1]
    te_flat = te.reshape(NT)

    nw, ch = 32, 16
    xs = _make_dispatch(T, D, PAD, nw, ch)(x, ipos0, ipos1)

    grid_spec = pltpu.PrefetchScalarGridSpec(
        num_scalar_prefetch=1,
        grid=(NT,),
        in_specs=[
            pl.BlockSpec((TMG, D), lambda i, te_r: (i, 0)),
            pl.BlockSpec((1, D, H), lambda i, te_r: (te_r[i], 0, 0)),
            pl.BlockSpec((1, D, H), lambda i, te_r: (te_r[i], 0, 0)),
            pl.BlockSpec((1, H, D), lambda i, te_r: (te_r[i], 0, 0)),
        ],
        out_specs=pl.BlockSpec((TMG, D), lambda i, te_r: (i, 0)),
    )
    os_rows = pl.pallas_call(
        _gmm_body,
        grid_spec=grid_spec,
        out_shape=jax.ShapeDtypeStruct((PAD, D), jnp.float32),
    )(te_flat, xs, gate_proj, up_proj, down_proj)

    oc0, oc1 = _make_combine(T, D, PAD, nw, ch)(os_rows, ipos0, ipos1)

    TMF = min(T, 512)
    SC = min(SH, 512)
    tf2, sh2 = T // TMF, SH // SC
    ffn = pl.pallas_call(
        _ffn_body,
        grid=(tf2, sh2),
        in_specs=[
            pl.BlockSpec((TMF, D), lambda t, s: (t, 0)),
            pl.BlockSpec((SC, D), lambda t, s: (s, 0)),
            pl.BlockSpec((SC, D), lambda t, s: (s, 0)),
            pl.BlockSpec((D, SC), lambda t, s: (0, s)),
        ],
        out_specs=pl.BlockSpec((TMF, D), lambda t, s: (t, 0)),
        out_shape=jax.ShapeDtypeStruct((T, D), jnp.float32),
    )(x, w1, w3, w2)

    s0c = scores[:, 0:1]
    s1c = scores[:, 1:2]
    TMS = min(T, 512)
    out = pl.pallas_call(
        _scale_body,
        grid=(T // TMS,),
        in_specs=[
            pl.BlockSpec((TMS, D), lambda t: (t, 0)),
            pl.BlockSpec((TMS, D), lambda t: (t, 0)),
            pl.BlockSpec((TMS, 1), lambda t: (t, 0)),
            pl.BlockSpec((TMS, 1), lambda t: (t, 0)),
            pl.BlockSpec((TMS, D), lambda t: (t, 0)),
        ],
        out_specs=pl.BlockSpec((TMS, D), lambda t: (t, 0)),
        out_shape=jax.ShapeDtypeStruct((T, D), jnp.float32),
    )(oc0, oc1, s0c, s1c, ffn)

    return out
